# baseline (speedup 1.0000x reference)
import functools

import jax
import jax.numpy as jnp
from jax import lax
from jax.experimental import pallas as pl
from jax.experimental.pallas import tpu as pltpu
from jax.experimental.pallas import tpu_sc as plsc

NUM_NODES = 10000
NUM_EDGES = 320000
HIDDEN = 128
LANES = 16
NW = 32
PER_W = NUM_EDGES // NW
BLK = 80
NBLK = PER_W // BLK
GRP = BLK // LANES


def _neg_sqrt(x):
    x = jnp.maximum(x, jnp.float32(1e-30))
    i = lax.bitcast_convert_type(x, jnp.int32)
    i = jnp.int32(0x5F3759DF) - (i >> 1)
    y = lax.bitcast_convert_type(i, jnp.float32)
    for _ in range(3):
        y = y * (jnp.float32(1.5) - jnp.float32(0.5) * x * y * y)
    return -(x * y)


def _body(z_hbm, src_hbm, dst_hbm, et_hbm, rel_hbm, out_hbm,
          ib, sA, tA, rA, sB, tB, rB, scr, oA, oB, zs, rs,
          semA, semB, semOA, semOB, semI):
    c = lax.axis_index("c")
    s = lax.axis_index("s")
    wid = s * 2 + c
    base = wid * PER_W

    lane = lax.iota(jnp.int32, LANES)
    col = lane * LANES

    ci1 = pltpu.async_copy(src_hbm.at[pl.ds(base, PER_W)],
                           ib.at[pl.ds(0, PER_W)], semI)
    ci2 = pltpu.async_copy(dst_hbm.at[pl.ds(base, PER_W)],
                           ib.at[pl.ds(PER_W, PER_W)], semI)
    ci3 = pltpu.async_copy(et_hbm.at[pl.ds(base, PER_W)],
                           ib.at[pl.ds(2 * PER_W, PER_W)], semI)
    zrows = NUM_NODES // LANES
    cz = pltpu.async_copy(z_hbm.at[pl.ds(s * zrows, zrows)],
                          zs.at[pl.ds(s * zrows, zrows)], semI)
    rrows = 1000 // LANES
    cr = pltpu.async_copy(rel_hbm.at[pl.ds(s * rrows, rrows)],
                          rs.at[pl.ds(s * rrows, rrows)], semI)

    @pl.when(s == 0)
    def _():
        pltpu.sync_copy(rel_hbm.at[pl.ds(LANES * rrows, 1000 - LANES * rrows)],
                        rs.at[pl.ds(LANES * rrows, 1000 - LANES * rrows)])

    ci1.wait()
    ci2.wait()
    ci3.wait()
    cz.wait()
    cr.wait()
    plsc.subcore_barrier()

    def issue(b, sbuf, tbuf, rbuf, sem):
        boff = b * BLK
        c1 = pltpu.async_copy(zs.at[ib.at[pl.ds(boff, BLK)]], sbuf, sem)
        c2 = pltpu.async_copy(zs.at[ib.at[pl.ds(PER_W + boff, BLK)]],
                              tbuf, sem)
        c3 = pltpu.async_copy(rs.at[ib.at[pl.ds(2 * PER_W + boff, BLK)]],
                              rbuf, sem)
        return c1, c2, c3

    def drain(cps):
        for cp in cps:
            cp.wait()

    def compute(sbuf, tbuf, rbuf, obuf):
        def grp(g):
            accs = [None] * LANES
            for ch in range(HIDDEN // 32):
                cs = pl.ds(ch * LANES, LANES)
                for e in range(LANES):
                    row = g * LANES + e
                    sv = plsc.bitcast(sbuf[row, cs], jnp.bfloat16)
                    rv = plsc.bitcast(rbuf[row, cs], jnp.bfloat16)
                    tv = plsc.bitcast(tbuf[row, cs], jnp.bfloat16)
                    d = sv + rv - tv
                    d0, d1 = plsc.unpack(d, format=plsc.PackFormat.INTERLEAVED)
                    sq = d0 * d0 + d1 * d1
                    accs[e] = sq if accs[e] is None else accs[e] + sq
            gb = g * LANES * LANES
            for e in range(LANES):
                scr[pl.ds(gb + e * LANES, LANES)] = accs[e]
            parts = [plsc.load_gather(scr, [col + (gb + l)])
                     for l in range(LANES)]
            while len(parts) > 1:
                parts = [a + b for a, b in zip(parts[::2], parts[1::2])]
            obuf[pl.ds(g * LANES, LANES)] = parts[0]

        for g in range(GRP):
            grp(g)
        vals = [obuf[pl.ds(g * LANES, LANES)] for g in range(GRP)]
        res = [_neg_sqrt(v) for v in vals]
        for g in range(GRP):
            obuf[pl.ds(g * LANES, LANES)] = res[g]

    def store(b, obuf, sem):
        return pltpu.async_copy(obuf, out_hbm.at[pl.ds(base + b * BLK, BLK)],
                                sem)

    def wait_store(obuf, sem):
        pltpu.make_async_copy(obuf, out_hbm.at[pl.ds(base, BLK)], sem).wait()

    drain(issue(0, sA, tA, rA, semA))

    def pair(k, carry):
        b0 = 2 * k
        b1 = b0 + 1
        cB = issue(b1, sB, tB, rB, semB)

        @pl.when(k > 0)
        def _():
            wait_store(oA, semOA)

        compute(sA, tA, rA, oA)
        store(b0, oA, semOA)
        cA = issue(b0 + 2, sA, tA, rA, semA)
        drain(cB)

        @pl.when(k > 0)
        def _():
            wait_store(oB, semOB)

        compute(sB, tB, rB, oB)
        store(b1, oB, semOB)
        drain(cA)
        return carry

    lax.fori_loop(0, (NBLK - 1) // 2, pair, 0)

    wait_store(oA, semOA)
    compute(sA, tA, rA, oA)
    store(NBLK - 1, oA, semOA)
    wait_store(oB, semOB)
    wait_store(oA, semOA)


@jax.jit
def _run(z, src, dst, et, rel_emb):
    mesh = plsc.VectorSubcoreMesh(core_axis_name="c", subcore_axis_name="s")
    f = functools.partial(
        pl.kernel,
        mesh=mesh,
        compiler_params=pltpu.CompilerParams(
            needs_layout_passes=False, use_tc_tiling_on_sc=False),
        out_type=jax.ShapeDtypeStruct((NUM_EDGES,), jnp.float32),
        scratch_types=[
            pltpu.VMEM((3 * PER_W,), jnp.int32),
            pltpu.VMEM((BLK, HIDDEN // 2), jnp.int32),
            pltpu.VMEM((BLK, HIDDEN // 2), jnp.int32),
            pltpu.VMEM((BLK, HIDDEN // 2), jnp.int32),
            pltpu.VMEM((BLK, HIDDEN // 2), jnp.int32),
            pltpu.VMEM((BLK, HIDDEN // 2), jnp.int32),
            pltpu.VMEM((BLK, HIDDEN // 2), jnp.int32),
            pltpu.VMEM((GRP * LANES * LANES,), jnp.float32),
            pltpu.VMEM((BLK,), jnp.float32),
            pltpu.VMEM((BLK,), jnp.float32),
            pltpu.VMEM_SHARED((NUM_NODES, HIDDEN // 2), jnp.int32),
            pltpu.VMEM_SHARED((1000, HIDDEN // 2), jnp.int32),
            pltpu.SemaphoreType.DMA,
            pltpu.SemaphoreType.DMA,
            pltpu.SemaphoreType.DMA,
            pltpu.SemaphoreType.DMA,
            pltpu.SemaphoreType.DMA,
        ],
    )(_body)
    return f(z, src, dst, et, rel_emb)


def kernel(z, edge_index, edge_type, rel_emb):
    src = edge_index[0].astype(jnp.int32)
    dst = edge_index[1].astype(jnp.int32)
    et = edge_type.astype(jnp.int32)
    zb = lax.bitcast_convert_type(
        z.astype(jnp.bfloat16).reshape(NUM_NODES, HIDDEN // 2, 2), jnp.int32)
    relb = lax.bitcast_convert_type(
        rel_emb.astype(jnp.bfloat16).reshape(-1, HIDDEN // 2, 2), jnp.int32)
    return _run(zb, src, dst, et, relb)


# speedup vs baseline: 1.5285x; 1.5285x over previous
import functools

import jax
import jax.numpy as jnp
from jax import lax
from jax.experimental import pallas as pl
from jax.experimental.pallas import tpu as pltpu
from jax.experimental.pallas import tpu_sc as plsc

NUM_NODES = 10000
NUM_EDGES = 320000
HIDDEN = 128
LANES = 16
NW = 32
PER_W = NUM_EDGES // NW
BLK = 80
NBLK = PER_W // BLK
GRP = BLK // LANES


def _neg_sqrt(x):
    x = jnp.maximum(x, jnp.float32(1e-30))
    i = lax.bitcast_convert_type(x, jnp.int32)
    i = jnp.int32(0x5F3759DF) - (i >> 1)
    y = lax.bitcast_convert_type(i, jnp.float32)
    for _ in range(3):
        y = y * (jnp.float32(1.5) - jnp.float32(0.5) * x * y * y)
    return -(x * y)


def _body(z_hbm, src_hbm, dst_hbm, et_hbm, rel_hbm, out_hbm,
          ib, sA, tA, rA, sB, tB, rB, scr, oA, oB, zs, rs,
          semA, semB, semOA, semOB, semI):
    c = lax.axis_index("c")
    s = lax.axis_index("s")
    wid = s * 2 + c
    base = wid * PER_W

    lane = lax.iota(jnp.int32, LANES)
    col = lane * LANES

    ci1 = pltpu.async_copy(src_hbm.at[pl.ds(base, PER_W)],
                           ib.at[pl.ds(0, PER_W)], semI)
    ci2 = pltpu.async_copy(dst_hbm.at[pl.ds(base, PER_W)],
                           ib.at[pl.ds(PER_W, PER_W)], semI)
    ci3 = pltpu.async_copy(et_hbm.at[pl.ds(base, PER_W)],
                           ib.at[pl.ds(2 * PER_W, PER_W)], semI)
    zrows = NUM_NODES // LANES
    cz = pltpu.async_copy(z_hbm.at[pl.ds(s * zrows, zrows)],
                          zs.at[pl.ds(s * zrows, zrows)], semI)
    rrows = 1000 // LANES
    cr = pltpu.async_copy(rel_hbm.at[pl.ds(s * rrows, rrows)],
                          rs.at[pl.ds(s * rrows, rrows)], semI)

    @pl.when(s == 0)
    def _():
        pltpu.sync_copy(rel_hbm.at[pl.ds(LANES * rrows, 1000 - LANES * rrows)],
                        rs.at[pl.ds(LANES * rrows, 1000 - LANES * rrows)])

    ci1.wait()
    ci2.wait()
    ci3.wait()
    cz.wait()
    cr.wait()
    plsc.subcore_barrier()

    def issue(b, sbuf, tbuf, rbuf, sem):
        boff = b * BLK
        c1 = pltpu.async_copy(zs.at[ib.at[pl.ds(boff, BLK)]], sbuf, sem)
        c2 = pltpu.async_copy(zs.at[ib.at[pl.ds(PER_W + boff, BLK)]],
                              tbuf, sem)
        c3 = pltpu.async_copy(rs.at[ib.at[pl.ds(2 * PER_W + boff, BLK)]],
                              rbuf, sem)
        return c1, c2, c3

    def drain(cps):
        for cp in cps:
            cp.wait()

    def compute(sbuf, tbuf, rbuf, obuf):
        def grp(g, gcarry):
            accs = [None] * LANES
            for ch in range(HIDDEN // 32):
                cs = pl.ds(ch * LANES, LANES)
                for e in range(LANES):
                    row = g * LANES + e
                    sv = plsc.bitcast(sbuf[row, cs], jnp.bfloat16)
                    rv = plsc.bitcast(rbuf[row, cs], jnp.bfloat16)
                    tv = plsc.bitcast(tbuf[row, cs], jnp.bfloat16)
                    d = sv + rv - tv
                    d0, d1 = plsc.unpack(d, format=plsc.PackFormat.INTERLEAVED)
                    sq = d0 * d0 + d1 * d1
                    accs[e] = sq if accs[e] is None else accs[e] + sq
            for e in range(LANES):
                scr[pl.ds(e * LANES, LANES)] = accs[e]
            parts = [plsc.load_gather(scr, [col + l]) for l in range(LANES)]
            while len(parts) > 1:
                parts = [a + b for a, b in zip(parts[::2], parts[1::2])]
            obuf[pl.ds(g * LANES, LANES)] = parts[0]
            return gcarry

        lax.fori_loop(0, GRP, grp, 0)
        vals = [obuf[pl.ds(g * LANES, LANES)] for g in range(GRP)]
        res = [_neg_sqrt(v) for v in vals]
        for g in range(GRP):
            obuf[pl.ds(g * LANES, LANES)] = res[g]

    def store(b, obuf, sem):
        return pltpu.async_copy(obuf, out_hbm.at[pl.ds(base + b * BLK, BLK)],
                                sem)

    def wait_store(obuf, sem):
        pltpu.make_async_copy(obuf, out_hbm.at[pl.ds(base, BLK)], sem).wait()

    drain(issue(0, sA, tA, rA, semA))

    def pair(k, carry):
        b0 = 2 * k
        b1 = b0 + 1
        cB = issue(b1, sB, tB, rB, semB)

        @pl.when(k > 0)
        def _():
            wait_store(oA, semOA)

        compute(sA, tA, rA, oA)
        store(b0, oA, semOA)
        cA = issue(b0 + 2, sA, tA, rA, semA)
        drain(cB)

        @pl.when(k > 0)
        def _():
            wait_store(oB, semOB)

        compute(sB, tB, rB, oB)
        store(b1, oB, semOB)
        drain(cA)
        return carry

    lax.fori_loop(0, (NBLK - 1) // 2, pair, 0)

    wait_store(oA, semOA)
    compute(sA, tA, rA, oA)
    store(NBLK - 1, oA, semOA)
    wait_store(oB, semOB)
    wait_store(oA, semOA)


@jax.jit
def _run(z, src, dst, et, rel_emb):
    mesh = plsc.VectorSubcoreMesh(core_axis_name="c", subcore_axis_name="s")
    f = functools.partial(
        pl.kernel,
        mesh=mesh,
        compiler_params=pltpu.CompilerParams(
            needs_layout_passes=False, use_tc_tiling_on_sc=False,
            skip_device_barrier=True),
        out_type=jax.ShapeDtypeStruct((NUM_EDGES,), jnp.float32),
        scratch_types=[
            pltpu.VMEM((3 * PER_W,), jnp.int32),
            pltpu.VMEM((BLK, HIDDEN // 2), jnp.int32),
            pltpu.VMEM((BLK, HIDDEN // 2), jnp.int32),
            pltpu.VMEM((BLK, HIDDEN // 2), jnp.int32),
            pltpu.VMEM((BLK, HIDDEN // 2), jnp.int32),
            pltpu.VMEM((BLK, HIDDEN // 2), jnp.int32),
            pltpu.VMEM((BLK, HIDDEN // 2), jnp.int32),
            pltpu.VMEM((LANES * LANES,), jnp.float32),
            pltpu.VMEM((BLK,), jnp.float32),
            pltpu.VMEM((BLK,), jnp.float32),
            pltpu.VMEM_SHARED((NUM_NODES, HIDDEN // 2), jnp.int32),
            pltpu.VMEM_SHARED((1000, HIDDEN // 2), jnp.int32),
            pltpu.SemaphoreType.DMA,
            pltpu.SemaphoreType.DMA,
            pltpu.SemaphoreType.DMA,
            pltpu.SemaphoreType.DMA,
            pltpu.SemaphoreType.DMA,
        ],
    )(_body)
    return f(z, src, dst, et, rel_emb)


def kernel(z, edge_index, edge_type, rel_emb):
    src = edge_index[0].astype(jnp.int32)
    dst = edge_index[1].astype(jnp.int32)
    et = edge_type.astype(jnp.int32)
    zb = lax.bitcast_convert_type(
        z.astype(jnp.bfloat16).reshape(NUM_NODES, HIDDEN // 2, 2), jnp.int32)
    relb = lax.bitcast_convert_type(
        rel_emb.astype(jnp.bfloat16).reshape(-1, HIDDEN // 2, 2), jnp.int32)
    return _run(zb, src, dst, et, relb)


# speedup vs baseline: 1.5303x; 1.0012x over previous
import functools

import jax
import jax.numpy as jnp
from jax import lax
from jax.experimental import pallas as pl
from jax.experimental.pallas import tpu as pltpu
from jax.experimental.pallas import tpu_sc as plsc

NUM_NODES = 10000
NUM_EDGES = 320000
HIDDEN = 128
LANES = 16
NW = 32
PER_W = NUM_EDGES // NW
BLK = 80
NBLK = PER_W // BLK
GRP = BLK // LANES


def _neg_sqrt(x):
    x = jnp.maximum(x, jnp.float32(1e-30))
    i = lax.bitcast_convert_type(x, jnp.int32)
    i = jnp.int32(0x5F3759DF) - (i >> 1)
    y = lax.bitcast_convert_type(i, jnp.float32)
    for _ in range(3):
        y = y * (jnp.float32(1.5) - jnp.float32(0.5) * x * y * y)
    return -(x * y)


def _body(z_hbm, src_hbm, dst_hbm, et_hbm, rel_hbm, out_hbm,
          ib, sA, tA, rA, sB, tB, rB, scr, oA, oB, zs, rs,
          semA, semB, semOA, semOB, semI):
    c = lax.axis_index("c")
    s = lax.axis_index("s")
    wid = s * 2 + c
    base = wid * PER_W

    lane = lax.iota(jnp.int32, LANES)
    col = lane * LANES

    ci1 = pltpu.async_copy(src_hbm.at[pl.ds(base, PER_W)],
                           ib.at[pl.ds(0, PER_W)], semI)
    ci2 = pltpu.async_copy(dst_hbm.at[pl.ds(base, PER_W)],
                           ib.at[pl.ds(PER_W, PER_W)], semI)
    ci3 = pltpu.async_copy(et_hbm.at[pl.ds(base, PER_W)],
                           ib.at[pl.ds(2 * PER_W, PER_W)], semI)
    zrows = NUM_NODES // LANES
    cz = pltpu.async_copy(z_hbm.at[pl.ds(s * zrows, zrows)],
                          zs.at[pl.ds(s * zrows, zrows)], semI)
    rrows = 1000 // LANES
    cr = pltpu.async_copy(rel_hbm.at[pl.ds(s * rrows, rrows)],
                          rs.at[pl.ds(s * rrows, rrows)], semI)

    @pl.when(s == 0)
    def _():
        pltpu.sync_copy(rel_hbm.at[pl.ds(LANES * rrows, 1000 - LANES * rrows)],
                        rs.at[pl.ds(LANES * rrows, 1000 - LANES * rrows)])

    ci1.wait()
    ci2.wait()
    ci3.wait()
    cz.wait()
    cr.wait()
    plsc.subcore_barrier()

    def issue(b, sbuf, tbuf, rbuf, sem):
        boff = b * BLK
        c1 = pltpu.async_copy(zs.at[ib.at[pl.ds(boff, BLK)]], sbuf, sem)
        c2 = pltpu.async_copy(zs.at[ib.at[pl.ds(PER_W + boff, BLK)]],
                              tbuf, sem)
        c3 = pltpu.async_copy(rs.at[ib.at[pl.ds(2 * PER_W + boff, BLK)]],
                              rbuf, sem)
        return c1, c2, c3

    def drain(cps):
        for cp in cps:
            cp.wait()

    def compute(sbuf, tbuf, rbuf, obuf):
        def grp(g, gcarry):
            accs = [None] * LANES
            for ch in range(HIDDEN // 32):
                cs = pl.ds(ch * LANES, LANES)
                for e in range(LANES):
                    row = g * LANES + e
                    sv = plsc.bitcast(sbuf[row, cs], jnp.bfloat16)
                    rv = plsc.bitcast(rbuf[row, cs], jnp.bfloat16)
                    tv = plsc.bitcast(tbuf[row, cs], jnp.bfloat16)
                    d = sv + rv - tv
                    d0, d1 = plsc.unpack(d, format=plsc.PackFormat.INTERLEAVED)
                    sq = d0 * d0 + d1 * d1
                    accs[e] = sq if accs[e] is None else accs[e] + sq
            for e in range(LANES):
                scr[pl.ds(e * LANES, LANES)] = accs[e]
            parts = [plsc.load_gather(scr, [col + l]) for l in range(LANES)]
            while len(parts) > 1:
                parts = [a + b for a, b in zip(parts[::2], parts[1::2])]
            obuf[pl.ds(g * LANES, LANES)] = parts[0]
            return gcarry

        lax.fori_loop(0, GRP, grp, 0)
        vals = [obuf[pl.ds(g * LANES, LANES)] for g in range(GRP)]
        res = [_neg_sqrt(v) for v in vals]
        for g in range(GRP):
            obuf[pl.ds(g * LANES, LANES)] = res[g]

    def store(b, obuf, sem):
        return pltpu.async_copy(obuf, out_hbm.at[pl.ds(base + b * BLK, BLK)],
                                sem)

    def wait_store(obuf, sem):
        pltpu.make_async_copy(obuf, out_hbm.at[pl.ds(base, BLK)], sem).wait()

    drain(issue(0, sA, tA, rA, semA))

    def pair(k, carry):
        b0 = 2 * k
        b1 = b0 + 1
        cB = issue(b1, sB, tB, rB, semB)

        @pl.when(k > 0)
        def _():
            wait_store(oA, semOA)

        compute(sA, tA, rA, oA)
        store(b0, oA, semOA)
        cA = issue(b0 + 2, sA, tA, rA, semA)
        drain(cB)

        @pl.when(k > 0)
        def _():
            wait_store(oB, semOB)

        compute(sB, tB, rB, oB)
        store(b1, oB, semOB)
        drain(cA)
        return carry

    lax.fori_loop(0, (NBLK - 1) // 2, pair, 0)

    wait_store(oA, semOA)
    compute(sA, tA, rA, oA)
    store(NBLK - 1, oA, semOA)
    wait_store(oB, semOB)
    wait_store(oA, semOA)


@jax.jit
def _run(z, src, dst, et, rel_emb):
    mesh = plsc.VectorSubcoreMesh(core_axis_name="c", subcore_axis_name="s")
    f = functools.partial(
        pl.kernel,
        mesh=mesh,
        compiler_params=pltpu.CompilerParams(
            needs_layout_passes=False, use_tc_tiling_on_sc=False),
        out_type=jax.ShapeDtypeStruct((NUM_EDGES,), jnp.float32),
        scratch_types=[
            pltpu.VMEM((3 * PER_W,), jnp.int32),
            pltpu.VMEM((BLK, HIDDEN // 2), jnp.int32),
            pltpu.VMEM((BLK, HIDDEN // 2), jnp.int32),
            pltpu.VMEM((BLK, HIDDEN // 2), jnp.int32),
            pltpu.VMEM((BLK, HIDDEN // 2), jnp.int32),
            pltpu.VMEM((BLK, HIDDEN // 2), jnp.int32),
            pltpu.VMEM((BLK, HIDDEN // 2), jnp.int32),
            pltpu.VMEM((LANES * LANES,), jnp.float32),
            pltpu.VMEM((BLK,), jnp.float32),
            pltpu.VMEM((BLK,), jnp.float32),
            pltpu.VMEM_SHARED((NUM_NODES, HIDDEN // 2), jnp.int32),
            pltpu.VMEM_SHARED((1000, HIDDEN // 2), jnp.int32),
            pltpu.SemaphoreType.DMA,
            pltpu.SemaphoreType.DMA,
            pltpu.SemaphoreType.DMA,
            pltpu.SemaphoreType.DMA,
            pltpu.SemaphoreType.DMA,
        ],
    )(_body)
    return f(z, src, dst, et, rel_emb)


def kernel(z, edge_index, edge_type, rel_emb):
    src = edge_index[0].astype(jnp.int32)
    dst = edge_index[1].astype(jnp.int32)
    et = edge_type.astype(jnp.int32)
    zb = lax.bitcast_convert_type(
        z.astype(jnp.bfloat16).reshape(NUM_NODES, HIDDEN // 2, 2), jnp.int32)
    relb = lax.bitcast_convert_type(
        rel_emb.astype(jnp.bfloat16).reshape(-1, HIDDEN // 2, 2), jnp.int32)
    return _run(zb, src, dst, et, relb)


# speedup vs baseline: 1.7867x; 1.1675x over previous
import functools

import jax
import jax.numpy as jnp
from jax import lax
from jax.experimental import pallas as pl
from jax.experimental.pallas import tpu as pltpu
from jax.experimental.pallas import tpu_sc as plsc

NUM_NODES = 10000
NUM_EDGES = 320000
HIDDEN = 128
LANES = 16
NW = 32
PER_W = NUM_EDGES // NW
BLK = 80
NBLK = PER_W // BLK
GRP = BLK // LANES


def _neg_sqrt(x):
    x = jnp.maximum(x, jnp.float32(1e-30))
    i = lax.bitcast_convert_type(x, jnp.int32)
    i = jnp.int32(0x5F3759DF) - (i >> 1)
    y = lax.bitcast_convert_type(i, jnp.float32)
    for _ in range(3):
        y = y * (jnp.float32(1.5) - jnp.float32(0.5) * x * y * y)
    return -(x * y)


def _body(z_hbm, src_hbm, dst_hbm, et_hbm, rel_hbm, out_hbm,
          ib, sA, tA, rA, sB, tB, rB, scr, oA, oB, zs, rs,
          semA, semB, semOA, semOB, semI):
    c = lax.axis_index("c")
    s = lax.axis_index("s")
    wid = s * 2 + c
    base = wid * PER_W

    lane = lax.iota(jnp.int32, LANES)
    col = lane * LANES

    ci1 = pltpu.async_copy(src_hbm.at[pl.ds(base, PER_W)],
                           ib.at[pl.ds(0, PER_W)], semI)
    ci2 = pltpu.async_copy(dst_hbm.at[pl.ds(base, PER_W)],
                           ib.at[pl.ds(PER_W, PER_W)], semI)
    ci3 = pltpu.async_copy(et_hbm.at[pl.ds(base, PER_W)],
                           ib.at[pl.ds(2 * PER_W, PER_W)], semI)
    zrows = NUM_NODES // LANES
    cz = pltpu.async_copy(z_hbm.at[pl.ds(s * zrows, zrows)],
                          zs.at[pl.ds(s * zrows, zrows)], semI)
    rrows = 1000 // LANES
    cr = pltpu.async_copy(rel_hbm.at[pl.ds(s * rrows, rrows)],
                          rs.at[pl.ds(s * rrows, rrows)], semI)

    @pl.when(s == 0)
    def _():
        pltpu.sync_copy(rel_hbm.at[pl.ds(LANES * rrows, 1000 - LANES * rrows)],
                        rs.at[pl.ds(LANES * rrows, 1000 - LANES * rrows)])

    ci1.wait()
    ci2.wait()
    ci3.wait()
    cz.wait()
    cr.wait()
    plsc.subcore_barrier()

    def issue(b, sbuf, tbuf, rbuf, sem):
        boff = b * BLK
        c1 = pltpu.async_copy(zs.at[ib.at[pl.ds(boff, BLK)]], sbuf, sem)
        c2 = pltpu.async_copy(zs.at[ib.at[pl.ds(PER_W + boff, BLK)]],
                              tbuf, sem)
        c3 = pltpu.async_copy(rs.at[ib.at[pl.ds(2 * PER_W + boff, BLK)]],
                              rbuf, sem)
        return c1, c2, c3

    def drain(cps):
        for cp in cps:
            cp.wait()

    def compute(sbuf, tbuf, rbuf, obuf):
        def grp(g, gcarry):
            accs = [None] * LANES
            for ch in range(HIDDEN // 32):
                cs = pl.ds(ch * LANES, LANES)
                for e in range(LANES):
                    row = g * LANES + e
                    sv = plsc.bitcast(sbuf[row, cs], jnp.bfloat16)
                    rv = plsc.bitcast(rbuf[row, cs], jnp.bfloat16)
                    tv = plsc.bitcast(tbuf[row, cs], jnp.bfloat16)
                    d = sv + rv - tv
                    d0, d1 = plsc.unpack(d, format=plsc.PackFormat.INTERLEAVED)
                    sq = d0 * d0 + d1 * d1
                    accs[e] = sq if accs[e] is None else accs[e] + sq
            for e in range(LANES):
                scr[pl.ds(e * LANES, LANES)] = accs[e]
            parts = [plsc.load_gather(scr, [col + l]) for l in range(LANES)]
            while len(parts) > 1:
                parts = [a + b for a, b in zip(parts[::2], parts[1::2])]
            obuf[pl.ds(g * LANES, LANES)] = parts[0]
            return gcarry

        lax.fori_loop(0, GRP, grp, 0)
        vals = [obuf[pl.ds(g * LANES, LANES)] for g in range(GRP)]
        res = [_neg_sqrt(v) for v in vals]
        for g in range(GRP):
            obuf[pl.ds(g * LANES, LANES)] = res[g]

    def store(b, obuf, sem):
        return pltpu.async_copy(obuf, out_hbm.at[pl.ds(base + b * BLK, BLK)],
                                sem)

    def wait_store(obuf, sem):
        pltpu.make_async_copy(obuf, out_hbm.at[pl.ds(base, BLK)], sem).wait()

    drain(issue(0, sA, tA, rA, semA))

    def pair(k, carry):
        b0 = 2 * k
        b1 = b0 + 1
        cB = issue(b1, sB, tB, rB, semB)

        @pl.when(k > 0)
        def _():
            wait_store(oA, semOA)

        compute(sA, tA, rA, oA)
        store(b0, oA, semOA)
        cA = issue(b0 + 2, sA, tA, rA, semA)
        drain(cB)

        @pl.when(k > 0)
        def _():
            wait_store(oB, semOB)

        compute(sB, tB, rB, oB)
        store(b1, oB, semOB)
        drain(cA)
        return carry

    lax.fori_loop(0, (NBLK - 1) // 2, pair, 0)

    wait_store(oA, semOA)
    compute(sA, tA, rA, oA)
    store(NBLK - 1, oA, semOA)
    wait_store(oB, semOB)
    wait_store(oA, semOA)


@jax.jit
def _run(z, src, dst, et, rel_emb):
    mesh = plsc.VectorSubcoreMesh(core_axis_name="c", subcore_axis_name="s")
    f = functools.partial(
        pl.kernel,
        mesh=mesh,
        compiler_params=pltpu.CompilerParams(
            needs_layout_passes=False, use_tc_tiling_on_sc=False),
        out_type=jax.ShapeDtypeStruct((NUM_EDGES,), jnp.float32),
        scratch_types=[
            pltpu.VMEM((3 * PER_W,), jnp.int32),
            pltpu.VMEM((BLK, HIDDEN // 2), jnp.int32),
            pltpu.VMEM((BLK, HIDDEN // 2), jnp.int32),
            pltpu.VMEM((BLK, HIDDEN // 2), jnp.int32),
            pltpu.VMEM((BLK, HIDDEN // 2), jnp.int32),
            pltpu.VMEM((BLK, HIDDEN // 2), jnp.int32),
            pltpu.VMEM((BLK, HIDDEN // 2), jnp.int32),
            pltpu.VMEM((LANES * LANES,), jnp.float32),
            pltpu.VMEM((BLK,), jnp.float32),
            pltpu.VMEM((BLK,), jnp.float32),
            pltpu.VMEM_SHARED((NUM_NODES, HIDDEN // 2), jnp.int32),
            pltpu.VMEM_SHARED((1000, HIDDEN // 2), jnp.int32),
            pltpu.SemaphoreType.DMA,
            pltpu.SemaphoreType.DMA,
            pltpu.SemaphoreType.DMA,
            pltpu.SemaphoreType.DMA,
            pltpu.SemaphoreType.DMA,
        ],
    )(_body)
    return f(z, src, dst, et, rel_emb)


def _pack_table(x):
    u = lax.bitcast_convert_type(x.astype(jnp.bfloat16), jnp.uint16)
    lo = u[:, :HIDDEN // 2].astype(jnp.uint32)
    hi = u[:, HIDDEN // 2:].astype(jnp.uint32)
    return lax.bitcast_convert_type(lo | (hi << 16), jnp.int32)


def kernel(z, edge_index, edge_type, rel_emb):
    src = edge_index[0].astype(jnp.int32)
    dst = edge_index[1].astype(jnp.int32)
    et = edge_type.astype(jnp.int32)
    return _run(_pack_table(z), src, dst, et, _pack_table(rel_emb))
